# baseline (speedup 1.0000x reference)
import functools

import jax
import jax.numpy as jnp
from jax import lax
from jax.experimental import pallas as pl
from jax.experimental.pallas import tpu as pltpu
from jax.experimental.pallas import tpu_sc as plsc

N = 10000
E = 320000
D = 128
NC = 2
NS = 16
NW = NC * NS
EPW = E // NW
CHUNK = 100
NCHUNK = EPW // CHUNK
NBUF = 2
ROWS_PER_TILE = 640
NPAD = NS * ROWS_PER_TILE


def _make_agg():
  mesh = plsc.VectorSubcoreMesh(core_axis_name="c", subcore_axis_name="s")

  @functools.partial(
      pl.kernel,
      out_type=jax.ShapeDtypeStruct((NC, NPAD, D), jnp.float32),
      mesh=mesh,
      compiler_params=pltpu.CompilerParams(use_tc_tiling_on_sc=False),
      scratch_types=[
          pltpu.VMEM_SHARED((NPAD, D), jnp.float32),
          pltpu.VMEM((NCHUNK, CHUNK), jnp.int32),
          pltpu.VMEM((NCHUNK, CHUNK), jnp.int32),
          pltpu.VMEM((NBUF, CHUNK, D), jnp.float32),
          [pltpu.SemaphoreType.DMA] * NBUF,
          [pltpu.SemaphoreType.DMA] * NBUF,
          pltpu.SemaphoreType.DMA,
      ],
  )
  def agg(x_hbm, src_hbm, dst_hbm, zeros_hbm, out_hbm, acc, src_v, dst_v,
          rows_v, sems, ssems, zsem):
    c = lax.axis_index("c")
    s = lax.axis_index("s")
    wid = s * NC + c
    pltpu.sync_copy(src_hbm.at[wid], src_v)
    zero_cp = pltpu.async_copy(
        zeros_hbm, acc.at[pl.ds(s * ROWS_PER_TILE, ROWS_PER_TILE)], zsem)
    for b in range(NBUF):
      pltpu.async_copy(x_hbm.at[src_v.at[b]], rows_v.at[b], sems[b])
    pltpu.sync_copy(dst_hbm.at[wid], dst_v)
    zero_cp.wait()
    plsc.subcore_barrier()

    @pl.loop(0, NCHUNK, step=NBUF)
    def _(g):
      for b in range(NBUF):
        c2 = g + b
        b2 = (b + NBUF - 1) % NBUF
        pltpu.make_async_copy(
            x_hbm.at[src_v.at[c2]], rows_v.at[b], sems[b]).wait()
        pltpu.async_copy(rows_v.at[b], acc.at[dst_v.at[c2]], ssems[b],
                         add=True)

        @pl.when(jnp.logical_and(c2 >= 1, c2 < NCHUNK - NBUF + 1))
        def _():
          pltpu.make_async_copy(
              rows_v.at[b2], acc.at[dst_v.at[c2 - 1]], ssems[b2]).wait()
          pltpu.async_copy(
              x_hbm.at[src_v.at[c2 - 1 + NBUF]], rows_v.at[b2], sems[b2])

    for k in range(NBUF):
      c2 = NCHUNK - NBUF + k
      pltpu.make_async_copy(
          rows_v.at[c2 % NBUF], acc.at[dst_v.at[c2]],
          ssems[c2 % NBUF]).wait()
    plsc.subcore_barrier()
    pltpu.sync_copy(
        acc.at[pl.ds(s * ROWS_PER_TILE, ROWS_PER_TILE)],
        out_hbm.at[c, pl.ds(s * ROWS_PER_TILE, ROWS_PER_TILE)])

  return agg


_BLK = 2000


def _mlp_body(x_ref, p_ref, wa_ref, ba_ref, wb_ref, bb_ref, o_ref):
  h = x_ref[...] + p_ref[0] + p_ref[1]
  h = jnp.dot(h, wa_ref[...], preferred_element_type=jnp.float32) + ba_ref[...]
  h = jnp.maximum(h, 0.0)
  o_ref[...] = (
      jnp.dot(h, wb_ref[...], preferred_element_type=jnp.float32) + bb_ref[...])


def _mlp(x, p, Wa, ba, Wb, bb):
  return pl.pallas_call(
      _mlp_body,
      grid=(N // _BLK,),
      in_specs=[
          pl.BlockSpec((_BLK, D), lambda i: (i, 0)),
          pl.BlockSpec((NC, _BLK, D), lambda i: (0, i, 0)),
          pl.BlockSpec((D, D), lambda i: (0, 0)),
          pl.BlockSpec((1, D), lambda i: (0, 0)),
          pl.BlockSpec((D, D), lambda i: (0, 0)),
          pl.BlockSpec((1, D), lambda i: (0, 0)),
      ],
      out_specs=pl.BlockSpec((_BLK, D), lambda i: (i, 0)),
      out_shape=jax.ShapeDtypeStruct((N, D), jnp.float32),
  )(x, p, Wa, ba, Wb, bb)


def kernel(x, edge_index, W1a, b1a, W1b, b1b, W2a, b2a, W2b, b2b):
  src = edge_index[0].reshape(NW, NCHUNK, CHUNK)
  dst = edge_index[1].reshape(NW, NCHUNK, CHUNK)
  zeros = jnp.zeros((ROWS_PER_TILE, D), jnp.float32)
  agg = _make_agg()
  p1 = agg(x, src, dst, zeros)
  h1 = _mlp(x, p1, W1a, b1a.reshape(1, D), W1b, b1b.reshape(1, D))
  p2 = agg(h1, src, dst, zeros)
  h2 = _mlp(h1, p2, W2a, b2a.reshape(1, D), W2b, b2b.reshape(1, D))
  return h2


# speedup vs baseline: 1.2177x; 1.2177x over previous
import functools

import jax
import jax.numpy as jnp
from jax import lax
from jax.experimental import pallas as pl
from jax.experimental.pallas import tpu as pltpu
from jax.experimental.pallas import tpu_sc as plsc

N = 10000
E = 320000
D = 128
NC = 2
NS = 16
NW = NC * NS
EPW = E // NW
CHUNK = 100
NCHUNK = EPW // CHUNK
NBUF = 2
ROWS_PER_TILE = 640
NPAD = NS * ROWS_PER_TILE


def _make_agg():
  mesh = plsc.VectorSubcoreMesh(core_axis_name="c", subcore_axis_name="s")

  @functools.partial(
      pl.kernel,
      out_type=jax.ShapeDtypeStruct((NC, NPAD, D), jnp.float32),
      mesh=mesh,
      compiler_params=pltpu.CompilerParams(use_tc_tiling_on_sc=False),
      scratch_types=[
          pltpu.VMEM_SHARED((NPAD, D), jnp.float32),
          pltpu.VMEM((NCHUNK, CHUNK), jnp.int32),
          pltpu.VMEM((NCHUNK, CHUNK), jnp.int32),
          pltpu.VMEM((NBUF, CHUNK, D), jnp.float32),
          [pltpu.SemaphoreType.DMA] * NBUF,
          pltpu.SemaphoreType.DMA,
      ],
  )
  def agg(x_hbm, src_hbm, dst_hbm, zeros_hbm, out_hbm, acc, src_v, dst_v,
          rows_v, sems, zsem):
    c = lax.axis_index("c")
    s = lax.axis_index("s")
    wid = s * NC + c
    pltpu.sync_copy(src_hbm.at[wid], src_v)
    zero_cp = pltpu.async_copy(
        zeros_hbm, acc.at[pl.ds(s * ROWS_PER_TILE, ROWS_PER_TILE)], zsem)
    for b in range(NBUF):
      pltpu.async_copy(x_hbm.at[src_v.at[b]], rows_v.at[b], sems[b])
    pltpu.sync_copy(dst_hbm.at[wid], dst_v)
    zero_cp.wait()
    plsc.subcore_barrier()

    @pl.loop(0, NCHUNK, step=NBUF)
    def _(g):
      for b in range(NBUF):
        pltpu.make_async_copy(
            x_hbm.at[src_v.at[g + b]], rows_v.at[b], sems[b]).wait()
        pltpu.sync_copy(rows_v.at[b], acc.at[dst_v.at[g + b]], add=True)

        @pl.when(g < NCHUNK - NBUF)
        def _():
          pltpu.async_copy(
              x_hbm.at[src_v.at[g + b + NBUF]], rows_v.at[b], sems[b])

    plsc.subcore_barrier()
    pltpu.sync_copy(
        acc.at[pl.ds(s * ROWS_PER_TILE, ROWS_PER_TILE)],
        out_hbm.at[c, pl.ds(s * ROWS_PER_TILE, ROWS_PER_TILE)])

  return agg


_BLK = 2000


def _mlp_body(x_ref, p_ref, wa_ref, ba_ref, wb_ref, bb_ref, o_ref):
  h = x_ref[...] + p_ref[0] + p_ref[1]
  h = jnp.dot(h, wa_ref[...], preferred_element_type=jnp.float32) + ba_ref[...]
  h = jnp.maximum(h, 0.0)
  o_ref[...] = (
      jnp.dot(h, wb_ref[...], preferred_element_type=jnp.float32) + bb_ref[...])


def _mlp(x, p, Wa, ba, Wb, bb):
  return pl.pallas_call(
      _mlp_body,
      grid=(N // _BLK,),
      in_specs=[
          pl.BlockSpec((_BLK, D), lambda i: (i, 0)),
          pl.BlockSpec((NC, _BLK, D), lambda i: (0, i, 0)),
          pl.BlockSpec((D, D), lambda i: (0, 0)),
          pl.BlockSpec((1, D), lambda i: (0, 0)),
          pl.BlockSpec((D, D), lambda i: (0, 0)),
          pl.BlockSpec((1, D), lambda i: (0, 0)),
      ],
      out_specs=pl.BlockSpec((_BLK, D), lambda i: (i, 0)),
      out_shape=jax.ShapeDtypeStruct((N, D), jnp.float32),
  )(x, p, Wa, ba, Wb, bb)


def kernel(x, edge_index, W1a, b1a, W1b, b1b, W2a, b2a, W2b, b2b):
  src = edge_index[0].reshape(NW, NCHUNK, CHUNK)
  dst = edge_index[1].reshape(NW, NCHUNK, CHUNK)
  zeros = jnp.zeros((ROWS_PER_TILE, D), jnp.float32)
  agg = _make_agg()
  p1 = agg(x, src, dst, zeros)
  h1 = _mlp(x, p1, W1a, b1a.reshape(1, D), W1b, b1b.reshape(1, D))
  p2 = agg(h1, src, dst, zeros)
  h2 = _mlp(h1, p2, W2a, b2a.reshape(1, D), W2b, b2b.reshape(1, D))
  return h2


# speedup vs baseline: 1.2625x; 1.0368x over previous
import functools

import jax
import jax.numpy as jnp
from jax import lax
from jax.experimental import pallas as pl
from jax.experimental.pallas import tpu as pltpu
from jax.experimental.pallas import tpu_sc as plsc

N = 10000
E = 320000
D = 128
NC = 2
NS = 16
NW = NC * NS
EPW = E // NW
CHUNK = 125
NCHUNK = EPW // CHUNK
NBUF = 2
IBUF = 4
ROWS_PER_TILE = 640
NPAD = NS * ROWS_PER_TILE


def _make_agg():
  mesh = plsc.VectorSubcoreMesh(core_axis_name="c", subcore_axis_name="s")

  @functools.partial(
      pl.kernel,
      out_type=jax.ShapeDtypeStruct((NC, NPAD, D), jnp.float32),
      mesh=mesh,
      compiler_params=pltpu.CompilerParams(use_tc_tiling_on_sc=False),
      scratch_types=[
          pltpu.VMEM_SHARED((NPAD, D), jnp.float32),
          pltpu.VMEM((NCHUNK, CHUNK), jnp.int32),
          pltpu.VMEM((IBUF, CHUNK), jnp.int32),
          pltpu.VMEM((NBUF, CHUNK, D), jnp.float32),
          [pltpu.SemaphoreType.DMA] * NBUF,
          [pltpu.SemaphoreType.DMA] * IBUF,
          pltpu.SemaphoreType.DMA,
      ],
  )
  def agg(x_hbm, src_hbm, dst_hbm, zeros_hbm, out_hbm, acc, src_v, dst_v,
          rows_v, sems, isems, zsem):
    c = lax.axis_index("c")
    s = lax.axis_index("s")
    wid = s * NC + c
    pltpu.sync_copy(src_hbm.at[wid], src_v)
    zero_cp = pltpu.async_copy(
        zeros_hbm, acc.at[pl.ds(s * ROWS_PER_TILE, ROWS_PER_TILE)], zsem)
    for b in range(NBUF):
      pltpu.async_copy(x_hbm.at[src_v.at[b]], rows_v.at[b], sems[b])
    for k in range(IBUF):
      pltpu.async_copy(dst_hbm.at[wid, k], dst_v.at[k], isems[k])
    zero_cp.wait()
    plsc.subcore_barrier()

    @pl.loop(0, NCHUNK, step=IBUF)
    def _(g):
      for b in range(IBUF):
        i = g + b
        rb = b % NBUF
        pltpu.make_async_copy(
            x_hbm.at[src_v.at[i]], rows_v.at[rb], sems[rb]).wait()
        pltpu.make_async_copy(
            dst_hbm.at[wid, i], dst_v.at[b], isems[b]).wait()
        pltpu.sync_copy(rows_v.at[rb], acc.at[dst_v.at[b]], add=True)

        @pl.when(i < NCHUNK - IBUF)
        def _():
          pltpu.async_copy(dst_hbm.at[wid, i + IBUF], dst_v.at[b], isems[b])

        @pl.when(i < NCHUNK - NBUF)
        def _():
          pltpu.async_copy(
              x_hbm.at[src_v.at[i + NBUF]], rows_v.at[rb], sems[rb])

    plsc.subcore_barrier()
    pltpu.sync_copy(
        acc.at[pl.ds(s * ROWS_PER_TILE, ROWS_PER_TILE)],
        out_hbm.at[c, pl.ds(s * ROWS_PER_TILE, ROWS_PER_TILE)])

  return agg


_BLK = 2000


def _mlp_body(x_ref, p_ref, wa_ref, ba_ref, wb_ref, bb_ref, o_ref):
  h = x_ref[...] + p_ref[0] + p_ref[1]
  h = jnp.dot(h, wa_ref[...], preferred_element_type=jnp.float32) + ba_ref[...]
  h = jnp.maximum(h, 0.0)
  o_ref[...] = (
      jnp.dot(h, wb_ref[...], preferred_element_type=jnp.float32) + bb_ref[...])


def _mlp(x, p, Wa, ba, Wb, bb):
  return pl.pallas_call(
      _mlp_body,
      grid=(N // _BLK,),
      in_specs=[
          pl.BlockSpec((_BLK, D), lambda i: (i, 0)),
          pl.BlockSpec((NC, _BLK, D), lambda i: (0, i, 0)),
          pl.BlockSpec((D, D), lambda i: (0, 0)),
          pl.BlockSpec((1, D), lambda i: (0, 0)),
          pl.BlockSpec((D, D), lambda i: (0, 0)),
          pl.BlockSpec((1, D), lambda i: (0, 0)),
      ],
      out_specs=pl.BlockSpec((_BLK, D), lambda i: (i, 0)),
      out_shape=jax.ShapeDtypeStruct((N, D), jnp.float32),
  )(x, p, Wa, ba, Wb, bb)


def kernel(x, edge_index, W1a, b1a, W1b, b1b, W2a, b2a, W2b, b2b):
  src = edge_index[0].reshape(NW, NCHUNK, CHUNK)
  dst = edge_index[1].reshape(NW, NCHUNK, CHUNK)
  zeros = jnp.zeros((ROWS_PER_TILE, D), jnp.float32)
  agg = _make_agg()
  p1 = agg(x, src, dst, zeros)
  h1 = _mlp(x, p1, W1a, b1a.reshape(1, D), W1b, b1b.reshape(1, D))
  p2 = agg(h1, src, dst, zeros)
  h2 = _mlp(h1, p2, W2a, b2a.reshape(1, D), W2b, b2b.reshape(1, D))
  return h2


# speedup vs baseline: 1.3098x; 1.0375x over previous
import functools

import jax
import jax.numpy as jnp
from jax import lax
from jax.experimental import pallas as pl
from jax.experimental.pallas import tpu as pltpu
from jax.experimental.pallas import tpu_sc as plsc

N = 10000
E = 320000
D = 128
NC = 2
NS = 16
NW = NC * NS
EPW = E // NW
CHUNK = 125
NCHUNK = EPW // CHUNK
NBUF = 2
IBUF = 4
ZROWS = 32
ROWS_PER_TILE = 640
NPAD = NS * ROWS_PER_TILE


def _make_agg():
  mesh = plsc.VectorSubcoreMesh(core_axis_name="c", subcore_axis_name="s")

  @functools.partial(
      pl.kernel,
      out_type=jax.ShapeDtypeStruct((NC, NPAD, D), jnp.float32),
      mesh=mesh,
      compiler_params=pltpu.CompilerParams(use_tc_tiling_on_sc=False),
      scratch_types=[
          pltpu.VMEM_SHARED((NPAD, D), jnp.float32),
          pltpu.VMEM((NCHUNK, CHUNK), jnp.int32),
          pltpu.VMEM((IBUF, CHUNK), jnp.int32),
          pltpu.VMEM((NBUF, CHUNK, D), jnp.float32),
          pltpu.VMEM((ZROWS, D), jnp.float32),
          [pltpu.SemaphoreType.DMA] * NBUF,
          [pltpu.SemaphoreType.DMA] * IBUF,
          pltpu.SemaphoreType.DMA,
      ],
  )
  def agg(x_hbm, src_hbm, dst_hbm, out_hbm, acc, src_v, dst_v,
          rows_v, zbuf, sems, isems, zsem):
    c = lax.axis_index("c")
    s = lax.axis_index("s")
    wid = s * NC + c
    pltpu.sync_copy(src_hbm.at[wid], src_v)
    for r in range(ZROWS):
      for j in range(D // 16):
        zbuf[r, pl.ds(j * 16, 16)] = jnp.zeros((16,), jnp.float32)
    for t in range(ROWS_PER_TILE // ZROWS):
      pltpu.async_copy(
          zbuf, acc.at[pl.ds(s * ROWS_PER_TILE + t * ZROWS, ZROWS)], zsem)
    for b in range(NBUF):
      pltpu.async_copy(x_hbm.at[src_v.at[b]], rows_v.at[b], sems[b])
    for k in range(IBUF):
      pltpu.async_copy(dst_hbm.at[wid, k], dst_v.at[k], isems[k])
    for t in range(ROWS_PER_TILE // ZROWS):
      pltpu.make_async_copy(
          zbuf, acc.at[pl.ds(s * ROWS_PER_TILE + t * ZROWS, ZROWS)],
          zsem).wait()
    plsc.subcore_barrier()

    @pl.loop(0, NCHUNK, step=IBUF)
    def _(g):
      for b in range(IBUF):
        i = g + b
        rb = b % NBUF
        pltpu.make_async_copy(
            x_hbm.at[src_v.at[i]], rows_v.at[rb], sems[rb]).wait()
        pltpu.make_async_copy(
            dst_hbm.at[wid, i], dst_v.at[b], isems[b]).wait()
        pltpu.sync_copy(rows_v.at[rb], acc.at[dst_v.at[b]], add=True)

        @pl.when(i < NCHUNK - IBUF)
        def _():
          pltpu.async_copy(dst_hbm.at[wid, i + IBUF], dst_v.at[b], isems[b])

        @pl.when(i < NCHUNK - NBUF)
        def _():
          pltpu.async_copy(
              x_hbm.at[src_v.at[i + NBUF]], rows_v.at[rb], sems[rb])

    plsc.subcore_barrier()
    pltpu.sync_copy(
        acc.at[pl.ds(s * ROWS_PER_TILE, ROWS_PER_TILE)],
        out_hbm.at[c, pl.ds(s * ROWS_PER_TILE, ROWS_PER_TILE)])

  return agg


_BLK = 2000


def _mlp_body(x_ref, p_ref, wa_ref, ba_ref, wb_ref, bb_ref, o_ref):
  h = x_ref[...] + p_ref[0] + p_ref[1]
  h = jnp.dot(h, wa_ref[...], preferred_element_type=jnp.float32) + ba_ref[...]
  h = jnp.maximum(h, 0.0)
  o_ref[...] = (
      jnp.dot(h, wb_ref[...], preferred_element_type=jnp.float32) + bb_ref[...])


def _mlp(x, p, Wa, ba, Wb, bb):
  return pl.pallas_call(
      _mlp_body,
      grid=(N // _BLK,),
      in_specs=[
          pl.BlockSpec((_BLK, D), lambda i: (i, 0)),
          pl.BlockSpec((NC, _BLK, D), lambda i: (0, i, 0)),
          pl.BlockSpec((D, D), lambda i: (0, 0)),
          pl.BlockSpec((1, D), lambda i: (0, 0)),
          pl.BlockSpec((D, D), lambda i: (0, 0)),
          pl.BlockSpec((1, D), lambda i: (0, 0)),
      ],
      out_specs=pl.BlockSpec((_BLK, D), lambda i: (i, 0)),
      out_shape=jax.ShapeDtypeStruct((N, D), jnp.float32),
  )(x, p, Wa, ba, Wb, bb)


def kernel(x, edge_index, W1a, b1a, W1b, b1b, W2a, b2a, W2b, b2b):
  src = edge_index[0].reshape(NW, NCHUNK, CHUNK)
  dst = edge_index[1].reshape(NW, NCHUNK, CHUNK)
  agg = _make_agg()
  p1 = agg(x, src, dst)
  h1 = _mlp(x, p1, W1a, b1a.reshape(1, D), W1b, b1b.reshape(1, D))
  p2 = agg(h1, src, dst)
  h2 = _mlp(h1, p2, W2a, b2a.reshape(1, D), W2b, b2b.reshape(1, D))
  return h2


# speedup vs baseline: 1.3830x; 1.0559x over previous
import functools

import jax
import jax.numpy as jnp
from jax import lax
from jax.experimental import pallas as pl
from jax.experimental.pallas import tpu as pltpu
from jax.experimental.pallas import tpu_sc as plsc

N = 10000
E = 320000
D = 128
NC = 2
NS = 16
NW = NC * NS
EPW = E // NW
CHUNK = 100
NCHUNK = EPW // CHUNK
NBUF = 3
IBUF = 2
STEP = 6
TAIL = NCHUNK % STEP
MAIN = NCHUNK - TAIL
ROWS_PER_TILE = 640
NPAD = NS * ROWS_PER_TILE


def _make_agg():
  mesh = plsc.VectorSubcoreMesh(core_axis_name="c", subcore_axis_name="s")

  @functools.partial(
      pl.kernel,
      out_type=jax.ShapeDtypeStruct((NC, NPAD, D), jnp.float32),
      mesh=mesh,
      compiler_params=pltpu.CompilerParams(use_tc_tiling_on_sc=False),
      scratch_types=[
          pltpu.VMEM_SHARED((NPAD, D), jnp.float32),
          pltpu.VMEM((NCHUNK, CHUNK), jnp.int32),
          pltpu.VMEM((IBUF, CHUNK), jnp.int32),
          pltpu.VMEM((NBUF, CHUNK, D), jnp.float32),
          [pltpu.SemaphoreType.DMA] * NBUF,
          [pltpu.SemaphoreType.DMA] * IBUF,
          pltpu.SemaphoreType.DMA,
      ],
  )
  def agg(x_hbm, src_hbm, dst_hbm, zeros_hbm, out_hbm, acc, src_v, dst_v,
          rows_v, sems, isems, zsem):
    c = lax.axis_index("c")
    s = lax.axis_index("s")
    wid = s * NC + c
    pltpu.sync_copy(src_hbm.at[wid], src_v)
    zero_cp = pltpu.async_copy(
        zeros_hbm, acc.at[pl.ds(s * ROWS_PER_TILE, ROWS_PER_TILE)], zsem)
    for b in range(NBUF):
      pltpu.async_copy(x_hbm.at[src_v.at[b]], rows_v.at[b], sems[b])
    for k in range(IBUF):
      pltpu.async_copy(dst_hbm.at[wid, k], dst_v.at[k], isems[k])
    zero_cp.wait()
    plsc.subcore_barrier()

    def slot(i, b):
      rb = b % NBUF
      kb = b % IBUF
      pltpu.make_async_copy(
          x_hbm.at[src_v.at[i]], rows_v.at[rb], sems[rb]).wait()
      pltpu.make_async_copy(
          dst_hbm.at[wid, i], dst_v.at[kb], isems[kb]).wait()
      pltpu.sync_copy(rows_v.at[rb], acc.at[dst_v.at[kb]], add=True)
      return rb, kb

    @pl.loop(0, MAIN, step=STEP)
    def _(g):
      for b in range(STEP):
        i = g + b
        rb, kb = slot(i, b)
        pltpu.async_copy(dst_hbm.at[wid, i + IBUF], dst_v.at[kb], isems[kb])
        pltpu.async_copy(x_hbm.at[src_v.at[i + NBUF]], rows_v.at[rb], sems[rb])

    for b in range(TAIL):
      i = MAIN + b
      rb, kb = slot(i, b)
      if i + IBUF < NCHUNK:
        pltpu.async_copy(dst_hbm.at[wid, i + IBUF], dst_v.at[kb], isems[kb])
      if i + NBUF < NCHUNK:
        pltpu.async_copy(x_hbm.at[src_v.at[i + NBUF]], rows_v.at[rb], sems[rb])

    plsc.subcore_barrier()
    pltpu.sync_copy(
        acc.at[pl.ds(s * ROWS_PER_TILE, ROWS_PER_TILE)],
        out_hbm.at[c, pl.ds(s * ROWS_PER_TILE, ROWS_PER_TILE)])

  return agg


_BLK = 2000


def _mlp_body(x_ref, p_ref, wa_ref, ba_ref, wb_ref, bb_ref, o_ref):
  h = x_ref[...] + p_ref[0] + p_ref[1]
  h = jnp.dot(h, wa_ref[...], preferred_element_type=jnp.float32) + ba_ref[...]
  h = jnp.maximum(h, 0.0)
  o_ref[...] = (
      jnp.dot(h, wb_ref[...], preferred_element_type=jnp.float32) + bb_ref[...])


def _mlp(x, p, Wa, ba, Wb, bb):
  return pl.pallas_call(
      _mlp_body,
      grid=(N // _BLK,),
      in_specs=[
          pl.BlockSpec((_BLK, D), lambda i: (i, 0)),
          pl.BlockSpec((NC, _BLK, D), lambda i: (0, i, 0)),
          pl.BlockSpec((D, D), lambda i: (0, 0)),
          pl.BlockSpec((1, D), lambda i: (0, 0)),
          pl.BlockSpec((D, D), lambda i: (0, 0)),
          pl.BlockSpec((1, D), lambda i: (0, 0)),
      ],
      out_specs=pl.BlockSpec((_BLK, D), lambda i: (i, 0)),
      out_shape=jax.ShapeDtypeStruct((N, D), jnp.float32),
  )(x, p, Wa, ba, Wb, bb)


def kernel(x, edge_index, W1a, b1a, W1b, b1b, W2a, b2a, W2b, b2b):
  src = edge_index[0].reshape(NW, NCHUNK, CHUNK)
  dst = edge_index[1].reshape(NW, NCHUNK, CHUNK)
  zeros = jnp.zeros((ROWS_PER_TILE, D), jnp.float32)
  agg = _make_agg()
  p1 = agg(x, src, dst, zeros)
  h1 = _mlp(x, p1, W1a, b1a.reshape(1, D), W1b, b1b.reshape(1, D))
  p2 = agg(h1, src, dst, zeros)
  h2 = _mlp(h1, p2, W2a, b2a.reshape(1, D), W2b, b2b.reshape(1, D))
  return h2


# speedup vs baseline: 1.4160x; 1.0239x over previous
import functools

import jax
import jax.numpy as jnp
from jax import lax
from jax.experimental import pallas as pl
from jax.experimental.pallas import tpu as pltpu
from jax.experimental.pallas import tpu_sc as plsc

N = 10000
E = 320000
D = 128
NC = 2
NS = 16
NW = NC * NS
EPW = E // NW
CHUNK = 100
NCHUNK = EPW // CHUNK
NBUF = 3
IBUF = 2
STEP = 6
TAIL = NCHUNK % STEP
MAIN = NCHUNK - TAIL
STRIPE = 624
ZROWS = 16
NPAD = N


def _make_agg():
  mesh = plsc.VectorSubcoreMesh(core_axis_name="c", subcore_axis_name="s")

  @functools.partial(
      pl.kernel,
      out_type=jax.ShapeDtypeStruct((NC, NPAD, D), jnp.float32),
      mesh=mesh,
      compiler_params=pltpu.CompilerParams(use_tc_tiling_on_sc=False),
      scratch_types=[
          pltpu.VMEM_SHARED((NPAD, D), jnp.float32),
          pltpu.VMEM((NCHUNK, CHUNK), jnp.int32),
          pltpu.VMEM((IBUF, CHUNK), jnp.int32),
          pltpu.VMEM((NBUF, CHUNK, D), jnp.float32),
          pltpu.VMEM((ZROWS, D), jnp.float32),
          [pltpu.SemaphoreType.DMA] * NBUF,
          [pltpu.SemaphoreType.DMA] * IBUF,
          pltpu.SemaphoreType.DMA,
      ],
  )
  def agg(x_hbm, src_hbm, dst_hbm, zeros_hbm, out_hbm, acc, src_v, dst_v,
          rows_v, zbuf, sems, isems, zsem):
    c = lax.axis_index("c")
    s = lax.axis_index("s")
    wid = s * NC + c
    base = s * STRIPE
    pltpu.sync_copy(src_hbm.at[wid], src_v)
    pltpu.sync_copy(zeros_hbm, zbuf)
    for t in range(STRIPE // ZROWS):
      pltpu.async_copy(zbuf, acc.at[pl.ds(base + t * ZROWS, ZROWS)], zsem)

    @pl.when(s == NS - 1)
    def _():
      pltpu.async_copy(zbuf, acc.at[pl.ds(base + STRIPE, ZROWS)], zsem)

    for b in range(NBUF):
      pltpu.async_copy(x_hbm.at[src_v.at[b]], rows_v.at[b], sems[b])
    for k in range(IBUF):
      pltpu.async_copy(dst_hbm.at[wid, k], dst_v.at[k], isems[k])
    for t in range(STRIPE // ZROWS):
      pltpu.make_async_copy(
          zbuf, acc.at[pl.ds(base + t * ZROWS, ZROWS)], zsem).wait()

    @pl.when(s == NS - 1)
    def _():
      pltpu.make_async_copy(
          zbuf, acc.at[pl.ds(base + STRIPE, ZROWS)], zsem).wait()

    plsc.subcore_barrier()

    def slot(i, b):
      rb = b % NBUF
      kb = b % IBUF
      pltpu.make_async_copy(
          x_hbm.at[src_v.at[i]], rows_v.at[rb], sems[rb]).wait()
      pltpu.make_async_copy(
          dst_hbm.at[wid, i], dst_v.at[kb], isems[kb]).wait()
      pltpu.sync_copy(rows_v.at[rb], acc.at[dst_v.at[kb]], add=True)
      return rb, kb

    @pl.loop(0, MAIN, step=STEP)
    def _(g):
      for b in range(STEP):
        i = g + b
        rb, kb = slot(i, b)
        pltpu.async_copy(dst_hbm.at[wid, i + IBUF], dst_v.at[kb], isems[kb])
        pltpu.async_copy(x_hbm.at[src_v.at[i + NBUF]], rows_v.at[rb], sems[rb])

    for b in range(TAIL):
      i = MAIN + b
      rb, kb = slot(i, b)
      if i + IBUF < NCHUNK:
        pltpu.async_copy(dst_hbm.at[wid, i + IBUF], dst_v.at[kb], isems[kb])
      if i + NBUF < NCHUNK:
        pltpu.async_copy(x_hbm.at[src_v.at[i + NBUF]], rows_v.at[rb], sems[rb])

    plsc.subcore_barrier()
    pltpu.sync_copy(
        acc.at[pl.ds(base, STRIPE)], out_hbm.at[c, pl.ds(base, STRIPE)])

    @pl.when(s == NS - 1)
    def _():
      pltpu.sync_copy(
          acc.at[pl.ds(base + STRIPE, ZROWS)],
          out_hbm.at[c, pl.ds(base + STRIPE, ZROWS)])

  return agg


_BLK = 2000


def _mlp_body(x_ref, p_ref, wa_ref, ba_ref, wb_ref, bb_ref, o_ref):
  h = x_ref[...] + p_ref[0] + p_ref[1]
  h = jnp.dot(h, wa_ref[...], preferred_element_type=jnp.float32) + ba_ref[...]
  h = jnp.maximum(h, 0.0)
  o_ref[...] = (
      jnp.dot(h, wb_ref[...], preferred_element_type=jnp.float32) + bb_ref[...])


def _mlp(x, p, Wa, ba, Wb, bb):
  return pl.pallas_call(
      _mlp_body,
      grid=(N // _BLK,),
      in_specs=[
          pl.BlockSpec((_BLK, D), lambda i: (i, 0)),
          pl.BlockSpec((NC, _BLK, D), lambda i: (0, i, 0)),
          pl.BlockSpec((D, D), lambda i: (0, 0)),
          pl.BlockSpec((1, D), lambda i: (0, 0)),
          pl.BlockSpec((D, D), lambda i: (0, 0)),
          pl.BlockSpec((1, D), lambda i: (0, 0)),
      ],
      out_specs=pl.BlockSpec((_BLK, D), lambda i: (i, 0)),
      out_shape=jax.ShapeDtypeStruct((N, D), jnp.float32),
  )(x, p, Wa, ba, Wb, bb)


def kernel(x, edge_index, W1a, b1a, W1b, b1b, W2a, b2a, W2b, b2b):
  src = edge_index[0].reshape(NW, NCHUNK, CHUNK)
  dst = edge_index[1].reshape(NW, NCHUNK, CHUNK)
  zeros = jnp.zeros((ZROWS, D), jnp.float32)
  agg = _make_agg()
  p1 = agg(x, src, dst, zeros)
  h1 = _mlp(x, p1, W1a, b1a.reshape(1, D), W1b, b1b.reshape(1, D))
  p2 = agg(h1, src, dst, zeros)
  h2 = _mlp(h1, p2, W2a, b2a.reshape(1, D), W2b, b2b.reshape(1, D))
  return h2
